# baseline (speedup 1.0000x reference)
import functools

import jax
import jax.numpy as jnp
from jax import lax
from jax.experimental import pallas as pl
from jax.experimental.pallas import tpu as pltpu
from jax.experimental.pallas import tpu_sc as plsc

NC = 2
NS = 16
LANES = 16


def _sc_spmm(m0, rows0, cols0, vals0, rows1, cols1, vals1, zrows):
    two_n, kh = m0.shape
    n = two_n // 2
    _, nch, g = rows0.shape
    cp = 80
    nclr = n // cp
    jmax = (nclr + NS - 1) // NS
    nquad = (nch - 1) // 4
    assert nch == 4 * nquad + 1

    mesh = plsc.VectorSubcoreMesh(
        core_axis_name="c", subcore_axis_name="s",
        num_cores=NC, num_subcores=NS)

    @functools.partial(
        pl.kernel,
        out_type=jax.ShapeDtypeStruct((8 * n, kh), jnp.float32),
        mesh=mesh,
        scratch_types=[
            pltpu.VMEM_SHARED((n, kh), jnp.float32),
            pltpu.VMEM((2, g), jnp.int32),
            pltpu.VMEM((2, g), jnp.int32),
            pltpu.VMEM((2, g), jnp.int32),
            pltpu.VMEM((2, g), jnp.int32),
            pltpu.VMEM((4, g), jnp.int32),
            pltpu.VMEM((4, g), jnp.int32),
            pltpu.VMEM((4, g), jnp.float32),
            pltpu.VMEM((g, kh), jnp.float32),
            pltpu.VMEM((g, kh), jnp.float32),
            pltpu.VMEM((g, kh), jnp.float32),
            pltpu.VMEM((g, kh), jnp.float32),
            pltpu.SemaphoreType.DMA,
            pltpu.SemaphoreType.DMA,
            pltpu.SemaphoreType.DMA,
            pltpu.SemaphoreType.DMA,
            pltpu.SemaphoreType.DMA,
            pltpu.SemaphoreType.DMA,
            pltpu.SemaphoreType.DMA,
            pltpu.SemaphoreType.DMA,
            pltpu.SemaphoreType.DMA,
            pltpu.SemaphoreType.DMA,
            pltpu.SemaphoreType.DMA,
            pltpu.SemaphoreType.DMA,
            pltpu.SemaphoreType.DMA,
        ],
    )
    def spmm_kernel(m0_ref, r0_ref, c0_ref, va0_ref, r1_ref, c1_ref,
                    va1_ref, z_ref, f_ref, acc, mb0, mb1, mb2, mb3,
                    rbuf, gbuf, vbuf,
                    rw0, rw1, rw2, rw3, sm0, sm1, sm2, sm3,
                    sg0, sg1, sg2, sg3, ss0, ss1, ss2, ss3, smisc):
        c = lax.axis_index("c")
        s = lax.axis_index("s")
        mbufs = (mb0, mb1, mb2, mb3)
        rows = (rw0, rw1, rw2, rw3)
        sms = (sm0, sm1, sm2, sm3)
        sgs = (sg0, sg1, sg2, sg3)
        sss = (ss0, ss1, ss2, ss3)

        def clear_acc():
            for j in range(jmax):
                cid = s + NS * j

                @pl.when(cid < nclr)
                def _(cid=cid):
                    pltpu.async_copy(z_ref, acc.at[pl.ds(cid * cp, cp)],
                                     smisc)
            for j in range(jmax):
                cid = s + NS * j

                @pl.when(cid < nclr)
                def _(cid=cid):
                    pltpu.make_async_copy(
                        z_ref, acc.at[pl.ds(cid * cp, cp)], smisc).wait()

        def copy_out(dst_base, zero_after):
            for j in range(jmax):
                cid = s + NS * j

                @pl.when(cid < nclr)
                def _(cid=cid):
                    pltpu.async_copy(
                        acc.at[pl.ds(cid * cp, cp)],
                        f_ref.at[pl.ds(dst_base + cid * cp, cp)], smisc)
            for j in range(jmax):
                cid = s + NS * j

                @pl.when(cid < nclr)
                def _(cid=cid):
                    pltpu.make_async_copy(
                        acc.at[pl.ds(cid * cp, cp)],
                        f_ref.at[pl.ds(dst_base + cid * cp, cp)],
                        smisc).wait()
            if zero_after:
                for j in range(jmax):
                    cid = s + NS * j

                    @pl.when(cid < nclr)
                    def _(cid=cid):
                        pltpu.async_copy(
                            z_ref, acc.at[pl.ds(cid * cp, cp)], smisc)
                for j in range(jmax):
                    cid = s + NS * j

                    @pl.when(cid < nclr)
                    def _(cid=cid):
                        pltpu.make_async_copy(
                            z_ref, acc.at[pl.ds(cid * cp, cp)],
                            smisc).wait()

        def scale(p):
            def qbody(q, _):
                vv = vbuf[p, pl.ds(q * LANES, LANES)]

                def jbody(j4, vv):
                    for u in range(4):
                        j = j4 * 4 + u
                        vb = lax.gather(
                            vv, jnp.full((LANES, 1), j, jnp.int32),
                            lax.GatherDimensionNumbers(
                                offset_dims=(), collapsed_slice_dims=(0,),
                                start_index_map=(0,)),
                            slice_sizes=(1,),
                            mode=lax.GatherScatterMode.PROMISE_IN_BOUNDS)
                        ei = q * LANES + j
                        for k in range(kh // LANES):
                            ksl = pl.ds(k * LANES, LANES)
                            rows[p][ei, ksl] = rows[p][ei, ksl] * vb
                    return vv
                lax.fori_loop(0, LANES // 4, jbody, vv)
                return 0
            lax.fori_loop(0, g // LANES, qbody, 0)

        def spmm_round(rsrc, csrc, vsrc, src_ref, off, dst_base,
                       clear_first, zero_after):
            def issue_meta(p, ci):
                pltpu.async_copy(rsrc.at[s, ci], mbufs[p].at[0], sms[p])
                pltpu.async_copy(csrc.at[s, ci], mbufs[p].at[1], sms[p])

            def issue_vals(p, ci):
                pltpu.async_copy(vsrc.at[s, ci], vbuf.at[p], sms[p])

            def wait_mv(p):
                pltpu.make_async_copy(rsrc.at[s, 0], mbufs[p].at[0],
                                      sms[p]).wait()
                pltpu.make_async_copy(csrc.at[s, 0], mbufs[p].at[1],
                                      sms[p]).wait()
                pltpu.make_async_copy(vsrc.at[s, 0], vbuf.at[p],
                                      sms[p]).wait()

            def prep(p):
                for q in range(g // LANES):
                    sl = pl.ds(q * LANES, LANES)
                    rbuf[p, sl] = mbufs[p][0, sl]
                    gbuf[p, sl] = mbufs[p][1, sl] + off

            def issue_gather(p):
                pltpu.async_copy(src_ref.at[gbuf.at[p]], rows[p], sgs[p])

            def wait_gather(p):
                pltpu.make_async_copy(src_ref.at[gbuf.at[p]], rows[p],
                                      sgs[p]).wait()

            def issue_scatter(p):
                pltpu.async_copy(rows[p], acc.at[rbuf.at[p]], sss[p],
                                 add=True)

            def wait_scatter(p):
                pltpu.make_async_copy(rows[p], acc.at[rbuf.at[p]],
                                      sss[p]).wait()

            if clear_first:
                clear_acc()
            for p in range(4):
                issue_meta(p, p)
                issue_vals(p, p)
            for p in range(2):
                wait_mv(p)
                prep(p)
                issue_gather(p)
            plsc.subcore_barrier()

            def qstep(t, _):
                c4 = 4 * t
                for p, ci in ((2, c4 + 2), (3, c4 + 3)):
                    wait_mv(p)

                    @pl.when(t > 0)
                    def _(p=p):
                        wait_scatter(p)
                    prep(p)
                    issue_gather(p)
                issue_meta(0, c4 + 4)

                @pl.when(c4 + 5 < nch)
                def _():
                    issue_meta(1, c4 + 5)

                wait_gather(0)
                scale(0)
                issue_scatter(0)
                issue_vals(0, c4 + 4)
                wait_gather(1)
                scale(1)
                issue_scatter(1)

                @pl.when(c4 + 5 < nch)
                def _():
                    issue_vals(1, c4 + 5)

                wait_mv(0)
                wait_scatter(0)
                prep(0)
                issue_gather(0)

                @pl.when(c4 + 5 < nch)
                def _():
                    wait_mv(1)
                    wait_scatter(1)
                    prep(1)
                    issue_gather(1)

                @pl.when(c4 + 6 < nch)
                def _():
                    issue_meta(2, c4 + 6)

                @pl.when(c4 + 7 < nch)
                def _():
                    issue_meta(3, c4 + 7)

                wait_gather(2)
                scale(2)
                issue_scatter(2)

                @pl.when(c4 + 6 < nch)
                def _():
                    issue_vals(2, c4 + 6)
                wait_gather(3)
                scale(3)
                issue_scatter(3)

                @pl.when(c4 + 7 < nch)
                def _():
                    issue_vals(3, c4 + 7)
                return 0
            lax.fori_loop(0, nquad, qstep, 0)

            wait_gather(0)
            scale(0)
            issue_scatter(0)
            wait_scatter(0)
            wait_scatter(1)
            wait_scatter(2)
            wait_scatter(3)
            plsc.subcore_barrier()
            copy_out(dst_base, zero_after)
            plsc.subcore_barrier()

        c_n = c * n
        spmm_round(r0_ref, c0_ref, va0_ref, m0_ref, c_n, c_n,
                   True, True)
        spmm_round(r0_ref, c0_ref, va0_ref, f_ref, c_n, 2 * n + c_n,
                   False, True)
        spmm_round(r1_ref, c1_ref, va1_ref, m0_ref, c_n, 4 * n + c_n,
                   False, True)
        spmm_round(r1_ref, c1_ref, va1_ref, f_ref, 4 * n + c_n,
                   6 * n + c_n, False, False)

    return spmm_kernel(m0, rows0, cols0, vals0, rows1, cols1, vals1, zrows)


def _tc_matmul(m0r, fr, w5):
    b, n, kh = m0r.shape
    out = w5.shape[2]
    blk = 1000
    nb = n // blk

    def body(m_ref, g0_ref, g1_ref, g2_ref, g3_ref, w_ref, o_ref):
        acc = jnp.dot(m_ref[0], w_ref[0], preferred_element_type=jnp.float32)
        acc = acc + jnp.dot(g0_ref[0], w_ref[1],
                            preferred_element_type=jnp.float32)
        acc = acc + jnp.dot(g1_ref[0], w_ref[2],
                            preferred_element_type=jnp.float32)
        acc = acc + jnp.dot(g2_ref[0], w_ref[3],
                            preferred_element_type=jnp.float32)
        acc = acc + jnp.dot(g3_ref[0], w_ref[4],
                            preferred_element_type=jnp.float32)
        o_ref[0] = acc

    def fspec(f):
        return pl.BlockSpec((1, blk, kh), lambda bi, j, f=f: (2 * f + bi, j, 0))

    return pl.pallas_call(
        body,
        grid=(b, nb),
        in_specs=[
            pl.BlockSpec((1, blk, kh), lambda bi, j: (bi, j, 0)),
            fspec(0), fspec(1), fspec(2), fspec(3),
            pl.BlockSpec((5, kh, out), lambda bi, j: (0, 0, 0)),
        ],
        out_specs=pl.BlockSpec((1, blk, out), lambda bi, j: (bi, j, 0)),
        out_shape=jax.ShapeDtypeStruct((b, n, out), jnp.float32),
    )(m0r, fr, fr, fr, fr, w5)


def kernel(A0_indices, A0_values, A1_indices, A1_values, X, W):
    b, d, n = X.shape
    e = A0_values.shape[0]
    out_f = W.shape[1]
    g = 80
    nch = e // (NS * g)

    m0 = jnp.swapaxes(X, 1, 2).reshape(b * n, d)

    r0 = A0_indices[0].astype(jnp.int32).reshape(NS, nch, g)
    c0 = A0_indices[1].astype(jnp.int32).reshape(NS, nch, g)
    r1 = A1_indices[0].astype(jnp.int32).reshape(NS, nch, g)
    c1 = A1_indices[1].astype(jnp.int32).reshape(NS, nch, g)
    vals0 = A0_values.reshape(NS, nch, g)
    vals1 = A1_values.reshape(NS, nch, g)
    z = jnp.zeros((80, d), jnp.float32)

    f = _sc_spmm(m0, r0, c0, vals0, r1, c1, vals1, z)

    wr = W.reshape(d, 5, out_f).transpose(1, 0, 2)
    w5 = jnp.stack([wr[0] - wr[2] - wr[4], wr[1], 2.0 * wr[2],
                    wr[3], 2.0 * wr[4]])

    return _tc_matmul(m0.reshape(b, n, d), f.reshape(8, n, d), w5)


# speedup vs baseline: 1.0399x; 1.0399x over previous
import functools

import jax
import jax.numpy as jnp
from jax import lax
from jax.experimental import pallas as pl
from jax.experimental.pallas import tpu as pltpu
from jax.experimental.pallas import tpu_sc as plsc

NC = 2
NS = 16
LANES = 16


def _sc_spmm(m0, rows0, cols0, vals0, rows1, cols1, vals1, zrows):
    two_n, kh = m0.shape
    n = two_n // 2
    _, nch, g = rows0.shape
    cp = 200
    nclr = n // cp
    jmax = (nclr + NS - 1) // NS
    nquad = (nch - 1) // 4
    assert nch == 4 * nquad + 1

    mesh = plsc.VectorSubcoreMesh(
        core_axis_name="c", subcore_axis_name="s",
        num_cores=NC, num_subcores=NS)

    @functools.partial(
        pl.kernel,
        out_type=jax.ShapeDtypeStruct((8 * n, kh), jnp.float32),
        mesh=mesh,
        scratch_types=[
            pltpu.VMEM_SHARED((n, kh), jnp.float32),
            pltpu.VMEM((2, g), jnp.int32),
            pltpu.VMEM((2, g), jnp.int32),
            pltpu.VMEM((2, g), jnp.int32),
            pltpu.VMEM((2, g), jnp.int32),
            pltpu.VMEM((4, g), jnp.int32),
            pltpu.VMEM((4, g), jnp.int32),
            pltpu.VMEM((4, g), jnp.float32),
            pltpu.VMEM((g, kh), jnp.float32),
            pltpu.VMEM((g, kh), jnp.float32),
            pltpu.VMEM((g, kh), jnp.float32),
            pltpu.VMEM((g, kh), jnp.float32),
            pltpu.SemaphoreType.DMA,
            pltpu.SemaphoreType.DMA,
            pltpu.SemaphoreType.DMA,
            pltpu.SemaphoreType.DMA,
            pltpu.SemaphoreType.DMA,
            pltpu.SemaphoreType.DMA,
            pltpu.SemaphoreType.DMA,
            pltpu.SemaphoreType.DMA,
            pltpu.SemaphoreType.DMA,
            pltpu.SemaphoreType.DMA,
            pltpu.SemaphoreType.DMA,
            pltpu.SemaphoreType.DMA,
            pltpu.SemaphoreType.DMA,
        ],
    )
    def spmm_kernel(m0_ref, r0_ref, c0_ref, va0_ref, r1_ref, c1_ref,
                    va1_ref, z_ref, f_ref, acc, mb0, mb1, mb2, mb3,
                    rbuf, gbuf, vbuf,
                    rw0, rw1, rw2, rw3, sm0, sm1, sm2, sm3,
                    sg0, sg1, sg2, sg3, ss0, ss1, ss2, ss3, smisc):
        c = lax.axis_index("c")
        s = lax.axis_index("s")
        mbufs = (mb0, mb1, mb2, mb3)
        rows = (rw0, rw1, rw2, rw3)
        sms = (sm0, sm1, sm2, sm3)
        sgs = (sg0, sg1, sg2, sg3)
        sss = (ss0, ss1, ss2, ss3)

        def clear_acc():
            for j in range(jmax):
                cid = s + NS * j

                @pl.when(cid < nclr)
                def _(cid=cid):
                    pltpu.async_copy(z_ref, acc.at[pl.ds(cid * cp, cp)],
                                     smisc)
            for j in range(jmax):
                cid = s + NS * j

                @pl.when(cid < nclr)
                def _(cid=cid):
                    pltpu.make_async_copy(
                        z_ref, acc.at[pl.ds(cid * cp, cp)], smisc).wait()

        def copy_out(dst_base, zero_after):
            for j in range(jmax):
                cid = s + NS * j

                @pl.when(cid < nclr)
                def _(cid=cid):
                    pltpu.async_copy(
                        acc.at[pl.ds(cid * cp, cp)],
                        f_ref.at[pl.ds(dst_base + cid * cp, cp)], smisc)
            for j in range(jmax):
                cid = s + NS * j

                @pl.when(cid < nclr)
                def _(cid=cid):
                    pltpu.make_async_copy(
                        acc.at[pl.ds(cid * cp, cp)],
                        f_ref.at[pl.ds(dst_base + cid * cp, cp)],
                        smisc).wait()
            if zero_after:
                for j in range(jmax):
                    cid = s + NS * j

                    @pl.when(cid < nclr)
                    def _(cid=cid):
                        pltpu.async_copy(
                            z_ref, acc.at[pl.ds(cid * cp, cp)], smisc)
                for j in range(jmax):
                    cid = s + NS * j

                    @pl.when(cid < nclr)
                    def _(cid=cid):
                        pltpu.make_async_copy(
                            z_ref, acc.at[pl.ds(cid * cp, cp)],
                            smisc).wait()

        def scale(p):
            def qbody(q, _):
                vv = vbuf[p, pl.ds(q * LANES, LANES)]

                def jbody(j4, vv):
                    for u in range(4):
                        j = j4 * 4 + u
                        vb = lax.gather(
                            vv, jnp.full((LANES, 1), j, jnp.int32),
                            lax.GatherDimensionNumbers(
                                offset_dims=(), collapsed_slice_dims=(0,),
                                start_index_map=(0,)),
                            slice_sizes=(1,),
                            mode=lax.GatherScatterMode.PROMISE_IN_BOUNDS)
                        ei = q * LANES + j
                        for k in range(kh // LANES):
                            ksl = pl.ds(k * LANES, LANES)
                            rows[p][ei, ksl] = rows[p][ei, ksl] * vb
                    return vv
                lax.fori_loop(0, LANES // 4, jbody, vv)
                return 0
            lax.fori_loop(0, g // LANES, qbody, 0)

        def spmm_round(rsrc, csrc, vsrc, src_ref, off, dst_base,
                       clear_first, zero_after):
            def issue_meta(p, ci):
                pltpu.async_copy(rsrc.at[s, ci], mbufs[p].at[0], sms[p])
                pltpu.async_copy(csrc.at[s, ci], mbufs[p].at[1], sms[p])

            def issue_vals(p, ci):
                pltpu.async_copy(vsrc.at[s, ci], vbuf.at[p], sms[p])

            def wait_mv(p):
                pltpu.make_async_copy(rsrc.at[s, 0], mbufs[p].at[0],
                                      sms[p]).wait()
                pltpu.make_async_copy(csrc.at[s, 0], mbufs[p].at[1],
                                      sms[p]).wait()
                pltpu.make_async_copy(vsrc.at[s, 0], vbuf.at[p],
                                      sms[p]).wait()

            def prep(p):
                for q in range(g // LANES):
                    sl = pl.ds(q * LANES, LANES)
                    rbuf[p, sl] = mbufs[p][0, sl]
                    gbuf[p, sl] = mbufs[p][1, sl] + off

            def issue_gather(p):
                pltpu.async_copy(src_ref.at[gbuf.at[p]], rows[p], sgs[p])

            def wait_gather(p):
                pltpu.make_async_copy(src_ref.at[gbuf.at[p]], rows[p],
                                      sgs[p]).wait()

            def issue_scatter(p):
                pltpu.async_copy(rows[p], acc.at[rbuf.at[p]], sss[p],
                                 add=True)

            def wait_scatter(p):
                pltpu.make_async_copy(rows[p], acc.at[rbuf.at[p]],
                                      sss[p]).wait()

            if clear_first:
                clear_acc()
            for p in range(4):
                issue_meta(p, p)
                issue_vals(p, p)
            for p in range(2):
                wait_mv(p)
                prep(p)
                issue_gather(p)
            plsc.subcore_barrier()

            def qstep(t, _):
                c4 = 4 * t
                for p, ci in ((2, c4 + 2), (3, c4 + 3)):
                    wait_mv(p)

                    @pl.when(t > 0)
                    def _(p=p):
                        wait_scatter(p)
                    prep(p)
                    issue_gather(p)
                issue_meta(0, c4 + 4)

                @pl.when(c4 + 5 < nch)
                def _():
                    issue_meta(1, c4 + 5)

                wait_gather(0)
                scale(0)
                issue_scatter(0)
                issue_vals(0, c4 + 4)
                wait_gather(1)
                scale(1)
                issue_scatter(1)

                @pl.when(c4 + 5 < nch)
                def _():
                    issue_vals(1, c4 + 5)

                wait_mv(0)
                wait_scatter(0)
                prep(0)
                issue_gather(0)

                @pl.when(c4 + 5 < nch)
                def _():
                    wait_mv(1)
                    wait_scatter(1)
                    prep(1)
                    issue_gather(1)

                @pl.when(c4 + 6 < nch)
                def _():
                    issue_meta(2, c4 + 6)

                @pl.when(c4 + 7 < nch)
                def _():
                    issue_meta(3, c4 + 7)

                wait_gather(2)
                scale(2)
                issue_scatter(2)

                @pl.when(c4 + 6 < nch)
                def _():
                    issue_vals(2, c4 + 6)
                wait_gather(3)
                scale(3)
                issue_scatter(3)

                @pl.when(c4 + 7 < nch)
                def _():
                    issue_vals(3, c4 + 7)
                return 0
            lax.fori_loop(0, nquad, qstep, 0)

            wait_gather(0)
            scale(0)
            issue_scatter(0)
            wait_scatter(0)
            wait_scatter(1)
            wait_scatter(2)
            wait_scatter(3)
            plsc.subcore_barrier()
            copy_out(dst_base, zero_after)
            plsc.subcore_barrier()

        c_n = c * n
        spmm_round(r0_ref, c0_ref, va0_ref, m0_ref, c_n, c_n,
                   True, True)
        spmm_round(r0_ref, c0_ref, va0_ref, f_ref, c_n, 2 * n + c_n,
                   False, True)
        spmm_round(r1_ref, c1_ref, va1_ref, m0_ref, c_n, 4 * n + c_n,
                   False, True)
        spmm_round(r1_ref, c1_ref, va1_ref, f_ref, 4 * n + c_n,
                   6 * n + c_n, False, False)

    return spmm_kernel(m0, rows0, cols0, vals0, rows1, cols1, vals1, zrows)


def _tc_matmul(m0r, fr, w5):
    b, n, kh = m0r.shape
    out = w5.shape[2]
    blk = 1000
    nb = n // blk

    def body(m_ref, g0_ref, g1_ref, g2_ref, g3_ref, w_ref, o_ref):
        acc = jnp.dot(m_ref[0], w_ref[0], preferred_element_type=jnp.float32)
        acc = acc + jnp.dot(g0_ref[0], w_ref[1],
                            preferred_element_type=jnp.float32)
        acc = acc + jnp.dot(g1_ref[0], w_ref[2],
                            preferred_element_type=jnp.float32)
        acc = acc + jnp.dot(g2_ref[0], w_ref[3],
                            preferred_element_type=jnp.float32)
        acc = acc + jnp.dot(g3_ref[0], w_ref[4],
                            preferred_element_type=jnp.float32)
        o_ref[0] = acc

    def fspec(f):
        return pl.BlockSpec((1, blk, kh), lambda bi, j, f=f: (2 * f + bi, j, 0))

    return pl.pallas_call(
        body,
        grid=(b, nb),
        in_specs=[
            pl.BlockSpec((1, blk, kh), lambda bi, j: (bi, j, 0)),
            fspec(0), fspec(1), fspec(2), fspec(3),
            pl.BlockSpec((5, kh, out), lambda bi, j: (0, 0, 0)),
        ],
        out_specs=pl.BlockSpec((1, blk, out), lambda bi, j: (bi, j, 0)),
        out_shape=jax.ShapeDtypeStruct((b, n, out), jnp.float32),
    )(m0r, fr, fr, fr, fr, w5)


def kernel(A0_indices, A0_values, A1_indices, A1_values, X, W):
    b, d, n = X.shape
    e = A0_values.shape[0]
    out_f = W.shape[1]
    g = 80
    nch = e // (NS * g)

    m0 = jnp.swapaxes(X, 1, 2).reshape(b * n, d)

    r0 = A0_indices[0].astype(jnp.int32).reshape(NS, nch, g)
    c0 = A0_indices[1].astype(jnp.int32).reshape(NS, nch, g)
    r1 = A1_indices[0].astype(jnp.int32).reshape(NS, nch, g)
    c1 = A1_indices[1].astype(jnp.int32).reshape(NS, nch, g)
    vals0 = A0_values.reshape(NS, nch, g)
    vals1 = A1_values.reshape(NS, nch, g)
    z = jnp.zeros((200, d), jnp.float32)

    f = _sc_spmm(m0, r0, c0, vals0, r1, c1, vals1, z)

    wr = W.reshape(d, 5, out_f).transpose(1, 0, 2)
    w5 = jnp.stack([wr[0] - wr[2] - wr[4], wr[1], 2.0 * wr[2],
                    wr[3], 2.0 * wr[4]])

    return _tc_matmul(m0.reshape(b, n, d), f.reshape(8, n, d), w5)
